# baseline (speedup 1.0000x reference)
import jax
import jax.numpy as jnp
from jax.experimental import pallas as pl
from jax.experimental.pallas import tpu as pltpu

_NUM_CLASSES = 20
_IOU_T = 0.5
_NEG_RATIO = 3.0
_VXY = 0.1
_VWH = 0.2
_EPS16 = 9.765625e-4
_IMGS = 8
_GP = 56


def _ssd_body(anc_ref, preg_ref, pcls_ref, gt_ref, bd_ref, out_ref):
    A = anc_ref.shape[1]
    C = pcls_ref.shape[1]
    N = _IMGS

    anc = anc_ref[...]
    acx, acy, aw, ah = anc[0:1], anc[1:2], anc[2:3], anc[3:4]
    al = acx - aw * 0.5
    at = acy - ah * 0.5
    ar = acx + aw * 0.5
    ab = acy + ah * 0.5
    area_a = (ar - al) * (ab - at)
    r_xw = 1.0 / (aw * _VXY)
    r_yh = 1.0 / (ah * _VXY)
    r_w = 1.0 / aw
    r_h = 1.0 / ah

    best_rows = []
    selb_rows = []
    for i in range(N):
        gt = gt_ref[i]
        gl = gt[:, 0:1]
        gtp = gt[:, 1:2]
        gr = gt[:, 2:3]
        gb = gt[:, 3:4]
        glab = gt[:, 4:5]

        w = jnp.maximum(jnp.minimum(gr, ar) - jnp.maximum(gl, al), 0.0)
        h = jnp.maximum(jnp.minimum(gb, ab) - jnp.maximum(gtp, at), 0.0)
        inter = w * h
        area_g = (gr - gl) * (gb - gtp)
        area_g = jnp.where(glab > 0.0, area_g, 1e30)
        iou = inter / jnp.maximum(area_g + area_a - inter, 1e-8)

        best = jnp.max(iou, axis=0, keepdims=True)
        gidx = jax.lax.broadcasted_iota(jnp.int32, (_GP, A), 0)
        best_idx = jnp.min(jnp.where(iou == best, gidx, _GP), axis=0,
                           keepdims=True)
        sel = gidx == best_idx
        best_rows.append(best)
        selb_rows.append(jnp.where(sel, 1.0, 0.0))

    best8 = jnp.concatenate(best_rows, axis=0)
    mask8 = best8 >= _IOU_T
    selb = jnp.concatenate(selb_rows, axis=0).astype(jnp.bfloat16)

    m = jnp.dot(bd_ref[0], selb, preferred_element_type=jnp.float32)
    gcx = m[0 * N:1 * N] + m[5 * N:6 * N] + m[9 * N:10 * N]
    gcy = m[1 * N:2 * N] + m[6 * N:7 * N] + m[10 * N:11 * N]
    gw = m[2 * N:3 * N] + m[7 * N:8 * N] + m[11 * N:12 * N]
    gh = m[3 * N:4 * N] + m[8 * N:9 * N] + m[12 * N:13 * N]
    glabel8 = jnp.where(mask8, m[4 * N:5 * N], 0.0)

    tx = (gcx - acx) * r_xw
    ty = (gcy - acy) * r_yh
    tw = jnp.log(jnp.maximum(gw, 1e-6) * r_w) * (1.0 / _VWH)
    th = jnp.log(jnp.maximum(gh, 1e-6) * r_h) * (1.0 / _VWH)

    def huber(dv):
        adv = jnp.abs(dv)
        return jnp.where(adv < 1.0, 0.5 * dv * dv, adv - 0.5)

    sl18 = (huber(preg_ref[:, 0, :] - tx) + huber(preg_ref[:, 1, :] - ty)
            + huber(preg_ref[:, 2, :] - tw)
            + huber(preg_ref[:, 3, :] - th))

    pc3 = pcls_ref[...]
    mx8 = jnp.max(pc3, axis=1)
    lse8 = mx8 + jnp.log(jnp.sum(jnp.exp(pc3 - mx8[:, None, :]), axis=1))
    cidx3 = jax.lax.broadcasted_iota(jnp.int32, (N, C, A), 1)
    gli = glabel8.astype(jnp.int32)
    p_at8 = jnp.sum(jnp.where(cidx3 == gli[:, None, :], pc3, 0.0), axis=1)
    ce8 = lse8 - p_at8

    np8 = jnp.sum(jnp.where(mask8, 1.0, 0.0), axis=1, keepdims=True)
    sl1p8 = jnp.sum(jnp.where(mask8, sl18, 0.0), axis=1, keepdims=True)
    cep8 = jnp.sum(jnp.where(mask8, ce8, 0.0), axis=1, keepdims=True)

    v8 = jnp.maximum(jnp.where(mask8, 0.0, ce8), 0.0)
    vb8 = pltpu.bitcast(v8, jnp.int32)
    kf = _NEG_RATIO * jnp.maximum(np8, _EPS16)
    kfl = jnp.floor(kf)
    k8 = kfl + jnp.where(kf > kfl, 1.0, 0.0)

    def count_ge(cand):
        return jnp.sum(jnp.where(vb8 >= cand, 1.0, 0.0), axis=1,
                       keepdims=True)

    p8 = jnp.zeros((N, 1), jnp.int32)
    for lo in range(28, 0, -3):
        cnts = [count_ge(p8 | (j << lo)) for j in range(1, 8)]
        inc = jnp.zeros((N, 1), jnp.int32)
        for j in range(7, 0, -1):
            inc = jnp.where((inc == 0) & (cnts[j - 1] >= k8), j, inc)
        p8 = p8 | (inc << lo)
    c0 = p8 | 1
    p8 = jnp.where(count_ge(c0) >= k8, c0, p8)

    cnt_gt = jnp.sum(jnp.where(vb8 > p8, 1.0, 0.0), axis=1, keepdims=True)
    sum_gt = jnp.sum(jnp.where(vb8 > p8, v8, 0.0), axis=1, keepdims=True)
    t8 = jnp.max(jnp.where(vb8 <= p8, v8, 0.0), axis=1, keepdims=True)
    neg8 = sum_gt + (k8 - cnt_gt) * t8

    lane = jax.lax.broadcasted_iota(jnp.int32, (N, 8), 1)
    out_ref[:, 0, :] = (jnp.where(lane == 0, np8, 0.0)
                        + jnp.where(lane == 1, sl1p8, 0.0)
                        + jnp.where(lane == 2, cep8, 0.0)
                        + jnp.where(lane == 3, neg8, 0.0))


def kernel(preg, pcls, ancs_xywh, gboxes_ltrb, glabels):
    B, _, A = preg.shape
    C = pcls.shape[1]
    G = gboxes_ltrb.shape[1]
    anc_t = ancs_xywh.T
    gt = jnp.concatenate(
        [gboxes_ltrb, glabels[..., None].astype(jnp.float32)], axis=-1)
    gt = jnp.pad(gt, ((0, 0), (0, _GP - G), (0, 0)))

    cx = (gt[:, :, 0] + gt[:, :, 2]) * 0.5
    cy = (gt[:, :, 1] + gt[:, :, 3]) * 0.5
    bw = gt[:, :, 2] - gt[:, :, 0]
    bh = gt[:, :, 3] - gt[:, :, 1]
    terms = []
    for qv in (cx, cy, bw, bh):
        hif = qv.astype(jnp.bfloat16).astype(jnp.float32)
        r1 = qv - hif
        midf = r1.astype(jnp.bfloat16).astype(jnp.float32)
        lof = (r1 - midf).astype(jnp.bfloat16).astype(jnp.float32)
        terms.append((hif, midf, lof))
    tlist = ([t[0] for t in terms] + [gt[:, :, 4]] + [t[1] for t in terms]
             + [t[2] for t in terms])
    tq = jnp.stack(tlist, axis=1).reshape(B // _IMGS, _IMGS, 13, _GP)
    bd = jnp.einsum('giqp,ij->gqijp', tq, jnp.eye(_IMGS, dtype=jnp.float32))
    bd = bd.reshape(B // _IMGS, 13 * _IMGS, _IMGS * _GP).astype(jnp.bfloat16)

    out = pl.pallas_call(
        _ssd_body,
        grid=(B // _IMGS,),
        in_specs=[
            pl.BlockSpec((4, A), lambda b: (0, 0)),
            pl.BlockSpec((_IMGS, 4, A), lambda b: (b, 0, 0)),
            pl.BlockSpec((_IMGS, C, A), lambda b: (b, 0, 0)),
            pl.BlockSpec((_IMGS, _GP, 5), lambda b: (b, 0, 0)),
            pl.BlockSpec((1, 13 * _IMGS, _IMGS * _GP), lambda b: (b, 0, 0)),
        ],
        out_specs=pl.BlockSpec((_IMGS, 1, 8), lambda b: (b, 0, 0)),
        out_shape=jax.ShapeDtypeStruct((B, 1, 8), jnp.float32),
        compiler_params=pltpu.CompilerParams(
            dimension_semantics=("parallel",),
            vmem_limit_bytes=56 * 1024 * 1024,
        ),
    )(anc_t, preg, pcls, gt, bd)

    r = out[:, 0, :]
    n_pos = r[:, 0]
    l_box = r[:, 1].sum() / jnp.maximum(n_pos.sum(), 1.0)
    nums = jnp.maximum(n_pos, _EPS16)
    return l_box + (r[:, 2] / nums).mean() + (r[:, 3] / nums).mean()


# speedup vs baseline: 1.0047x; 1.0047x over previous
import jax
import jax.numpy as jnp
from jax.experimental import pallas as pl
from jax.experimental.pallas import tpu as pltpu

_NUM_CLASSES = 20
_IOU_T = 0.5
_NEG_RATIO = 3.0
_VXY = 0.1
_VWH = 0.2
_EPS16 = 9.765625e-4
_IMGS = 8
_GP = 56


def _ssd_body(anc_ref, preg_ref, pcls_ref, gt_ref, bd_ref, out_ref):
    A = anc_ref.shape[1]
    C = pcls_ref.shape[1]
    N = _IMGS

    anc = anc_ref[...]
    acx, acy, aw, ah = anc[0:1], anc[1:2], anc[2:3], anc[3:4]
    al = acx - aw * 0.5
    at = acy - ah * 0.5
    ar = acx + aw * 0.5
    ab = acy + ah * 0.5
    area_a = (ar - al) * (ab - at)
    r_xw = 1.0 / (aw * _VXY)
    r_yh = 1.0 / (ah * _VXY)
    r_w = 1.0 / aw
    r_h = 1.0 / ah

    best_rows = []
    selb_rows = []
    for i in range(N):
        gt = gt_ref[i]
        gl = gt[:, 0:1]
        gtp = gt[:, 1:2]
        gr = gt[:, 2:3]
        gb = gt[:, 3:4]
        glab = gt[:, 4:5]

        w = jnp.maximum(jnp.minimum(gr, ar) - jnp.maximum(gl, al), 0.0)
        h = jnp.maximum(jnp.minimum(gb, ab) - jnp.maximum(gtp, at), 0.0)
        inter = w * h
        area_g = (gr - gl) * (gb - gtp)
        area_g = jnp.where(glab > 0.0, area_g, 1e30)
        iou = inter / jnp.maximum(area_g + area_a - inter, 1e-8)

        best = jnp.max(iou, axis=0, keepdims=True)
        gidx = jax.lax.broadcasted_iota(jnp.int32, (_GP, A), 0)
        best_idx = jnp.min(jnp.where(iou == best, gidx, _GP), axis=0,
                           keepdims=True)
        sel = gidx == best_idx
        best_rows.append(best)
        selb_rows.append(jnp.where(sel, 1.0, 0.0))

    best8 = jnp.concatenate(best_rows, axis=0)
    mask8 = best8 >= _IOU_T
    selb = jnp.concatenate(selb_rows, axis=0).astype(jnp.bfloat16)

    m = jnp.dot(bd_ref[0], selb, preferred_element_type=jnp.float32)
    gcx = m[0 * N:1 * N] + m[5 * N:6 * N] + m[9 * N:10 * N]
    gcy = m[1 * N:2 * N] + m[6 * N:7 * N] + m[10 * N:11 * N]
    gw = m[2 * N:3 * N] + m[7 * N:8 * N] + m[11 * N:12 * N]
    gh = m[3 * N:4 * N] + m[8 * N:9 * N] + m[12 * N:13 * N]
    glabel8 = jnp.where(mask8, m[4 * N:5 * N], 0.0)

    tx = (gcx - acx) * r_xw
    ty = (gcy - acy) * r_yh
    tw = jnp.log(jnp.maximum(gw, 1e-6) * r_w) * (1.0 / _VWH)
    th = jnp.log(jnp.maximum(gh, 1e-6) * r_h) * (1.0 / _VWH)

    def huber(dv):
        adv = jnp.abs(dv)
        return jnp.where(adv < 1.0, 0.5 * dv * dv, adv - 0.5)

    sl18 = (huber(preg_ref[:, 0, :] - tx) + huber(preg_ref[:, 1, :] - ty)
            + huber(preg_ref[:, 2, :] - tw)
            + huber(preg_ref[:, 3, :] - th))

    pc3 = pcls_ref[...]
    mx8 = jnp.max(pc3, axis=1)
    lse8 = mx8 + jnp.log(jnp.sum(jnp.exp(pc3 - mx8[:, None, :]), axis=1))
    cidx3 = jax.lax.broadcasted_iota(jnp.int32, (N, C, A), 1)
    gli = glabel8.astype(jnp.int32)
    p_at8 = jnp.sum(jnp.where(cidx3 == gli[:, None, :], pc3, 0.0), axis=1)
    ce8 = lse8 - p_at8

    np8 = jnp.sum(jnp.where(mask8, 1.0, 0.0), axis=1, keepdims=True)
    sl1p8 = jnp.sum(jnp.where(mask8, sl18, 0.0), axis=1, keepdims=True)
    cep8 = jnp.sum(jnp.where(mask8, ce8, 0.0), axis=1, keepdims=True)

    v8 = jnp.maximum(jnp.where(mask8, 0.0, ce8), 0.0)
    vb8 = pltpu.bitcast(v8, jnp.int32)
    kf = _NEG_RATIO * jnp.maximum(np8, _EPS16)
    kfl = jnp.floor(kf)
    k8 = kfl + jnp.where(kf > kfl, 1.0, 0.0)

    def count_ge(cand):
        return jnp.sum(jnp.where(vb8 >= cand, 1.0, 0.0), axis=1,
                       keepdims=True)

    p8 = jnp.zeros((N, 1), jnp.int32)
    for lo in range(29, 0, -2):
        c1 = p8 | (1 << lo)
        c2 = p8 | (2 << lo)
        c3 = p8 | (3 << lo)
        n1, n2, n3 = count_ge(c1), count_ge(c2), count_ge(c3)
        inc = jnp.where(n3 >= k8, 3,
                        jnp.where(n2 >= k8, 2, jnp.where(n1 >= k8, 1, 0)))
        p8 = p8 | (inc << lo)
    c0 = p8 | 1
    p8 = jnp.where(count_ge(c0) >= k8, c0, p8)

    cnt_gt = jnp.sum(jnp.where(vb8 > p8, 1.0, 0.0), axis=1, keepdims=True)
    sum_gt = jnp.sum(jnp.where(vb8 > p8, v8, 0.0), axis=1, keepdims=True)
    t8 = jnp.max(jnp.where(vb8 <= p8, v8, 0.0), axis=1, keepdims=True)
    neg8 = sum_gt + (k8 - cnt_gt) * t8

    lane = jax.lax.broadcasted_iota(jnp.int32, (N, 8), 1)
    out_ref[:, 0, :] = (jnp.where(lane == 0, np8, 0.0)
                        + jnp.where(lane == 1, sl1p8, 0.0)
                        + jnp.where(lane == 2, cep8, 0.0)
                        + jnp.where(lane == 3, neg8, 0.0))


def kernel(preg, pcls, ancs_xywh, gboxes_ltrb, glabels):
    B, _, A = preg.shape
    C = pcls.shape[1]
    G = gboxes_ltrb.shape[1]
    anc_t = ancs_xywh.T
    gt = jnp.concatenate(
        [gboxes_ltrb, glabels[..., None].astype(jnp.float32)], axis=-1)
    gt = jnp.pad(gt, ((0, 0), (0, _GP - G), (0, 0)))

    cx = (gt[:, :, 0] + gt[:, :, 2]) * 0.5
    cy = (gt[:, :, 1] + gt[:, :, 3]) * 0.5
    bw = gt[:, :, 2] - gt[:, :, 0]
    bh = gt[:, :, 3] - gt[:, :, 1]
    terms = []
    for qv in (cx, cy, bw, bh):
        hif = qv.astype(jnp.bfloat16).astype(jnp.float32)
        r1 = qv - hif
        midf = r1.astype(jnp.bfloat16).astype(jnp.float32)
        lof = (r1 - midf).astype(jnp.bfloat16).astype(jnp.float32)
        terms.append((hif, midf, lof))
    tlist = ([t[0] for t in terms] + [gt[:, :, 4]] + [t[1] for t in terms]
             + [t[2] for t in terms])
    tq = jnp.stack(tlist, axis=1).reshape(B // _IMGS, _IMGS, 13, _GP)
    bd = jnp.einsum('giqp,ij->gqijp', tq, jnp.eye(_IMGS, dtype=jnp.float32))
    bd = bd.reshape(B // _IMGS, 13 * _IMGS, _IMGS * _GP).astype(jnp.bfloat16)

    out = pl.pallas_call(
        _ssd_body,
        grid=(B // _IMGS,),
        in_specs=[
            pl.BlockSpec((4, A), lambda b: (0, 0)),
            pl.BlockSpec((_IMGS, 4, A), lambda b: (b, 0, 0)),
            pl.BlockSpec((_IMGS, C, A), lambda b: (b, 0, 0)),
            pl.BlockSpec((_IMGS, _GP, 5), lambda b: (b, 0, 0)),
            pl.BlockSpec((1, 13 * _IMGS, _IMGS * _GP), lambda b: (b, 0, 0)),
        ],
        out_specs=pl.BlockSpec((_IMGS, 1, 8), lambda b: (b, 0, 0)),
        out_shape=jax.ShapeDtypeStruct((B, 1, 8), jnp.float32),
        compiler_params=pltpu.CompilerParams(
            dimension_semantics=("parallel",),
            vmem_limit_bytes=56 * 1024 * 1024,
        ),
    )(anc_t, preg, pcls, gt, bd)

    r = out[:, 0, :]
    n_pos = r[:, 0]
    l_box = r[:, 1].sum() / jnp.maximum(n_pos.sum(), 1.0)
    nums = jnp.maximum(n_pos, _EPS16)
    return l_box + (r[:, 2] / nums).mean() + (r[:, 3] / nums).mean()
